# Initial kernel scaffold; baseline (speedup 1.0000x reference)
import jax
import jax.numpy as jnp
from jax.experimental import pallas as pl


def kernel(x, coor, sparse_idx, Wq, bq, Wk, bk, W0, b0, W1, b1, gamma, beta, W2, b2):
    raise NotImplementedError("write your pallas kernel here")


# speedup vs baseline: 1.8864x; 1.8864x over previous
import functools

import jax
import jax.numpy as jnp
from jax import lax
from jax.experimental import pallas as pl
from jax.experimental.pallas import tpu as pltpu
from jax.experimental.pallas import tpu_sc as plsc

N = 4096
K = 32
E = N * K
LOC = 3
GEODESIC = 0.5

BE = 4096
NEB = E // BE
BN = 512
NRB = N // BN

NW = 32
ROWS_PER_W = N // NW
ZW = 16384
SLAB = ROWS_PER_W * N


def _mlp_t(c, w0t, b0, w1t, b1):
    h = jnp.tanh(c)
    h = jnp.dot(h, w0t, preferred_element_type=jnp.float32) + b0
    h = jnp.dot(h, w1t, preferred_element_type=jnp.float32) + b1
    return jnp.tanh(h)


def _stats_body(c_ref, w0_ref, b0_ref, w1_ref, b1_ref, o_ref):
    t = _mlp_t(c_ref[...], w0_ref[...], b0_ref[...], w1_ref[...], b1_ref[...])
    ps = jnp.sum(t, axis=0, keepdims=True)
    ps2 = jnp.sum(t * t, axis=0, keepdims=True)

    @pl.when(pl.program_id(0) == 0)
    def _():
        o_ref[...] = jnp.zeros_like(o_ref)

    o_ref[...] += jnp.concatenate([ps, ps2], axis=0)


def _lck_body(c_ref, w0_ref, b0_ref, w1_ref, b1_ref, s_ref, g_ref, be_ref,
              w2_ref, b2_ref, o_ref):
    t = _mlp_t(c_ref[...], w0_ref[...], b0_ref[...], w1_ref[...], b1_ref[...])
    inv_e = jnp.float32(1.0 / E)
    m = s_ref[0:1, :] * inv_e
    var = s_ref[1:2, :] * inv_e - m * m
    scale = g_ref[...] * lax.rsqrt(var + 1e-5)
    shift = be_ref[...] - m * scale
    g = jnp.tanh(t * scale + shift)
    lck = jnp.dot(g, w2_ref[...], preferred_element_type=jnp.float32) + b2_ref[...]
    o_ref[...] = jnp.maximum(lck, 0.0)


def _att_body(c6_ref, cols_ref, lck_ref, p_ref, val_ref, idx_ref):
    c = c6_ref[...]
    s0 = c[0] + c[3]
    s1 = c[1] + c[4]
    s2 = c[2] + c[5]
    ctr0 = s0[0:1, :]
    ctr1 = s1[0:1, :]
    ctr2 = s2[0:1, :]
    ca0 = ctr0 * p_ref[0] + ctr1 * p_ref[3] + ctr2 * p_ref[6] + p_ref[12]
    ca1 = ctr0 * p_ref[1] + ctr1 * p_ref[4] + ctr2 * p_ref[7] + p_ref[13]
    ca2 = ctr0 * p_ref[2] + ctr1 * p_ref[5] + ctr2 * p_ref[8] + p_ref[14]
    cu = ctr0 * p_ref[9] + ctr1 * p_ref[10] + ctr2 * p_ref[11] + p_ref[15]
    att = ca0 * s0 + ca1 * s1 + ca2 * s2 + cu
    alpha = jnp.abs(att)

    cols = cols_ref[...]
    lck = lck_ref[...]
    lsum = jnp.zeros_like(lck)
    asum = jnp.zeros_like(lck)
    for k2 in range(K):
        m = cols == cols[k2:k2 + 1, :]
        lsum += jnp.where(m, lck[k2:k2 + 1, :], 0.0)
        asum += jnp.where(m, alpha[k2:k2 + 1, :], 0.0)

    val_ref[...] = lsum * jnp.exp(asum * jnp.float32(-GEODESIC))
    row = (lax.broadcasted_iota(jnp.int32, (K, BN), 1)
           + pl.program_id(0) * BN)
    idx_ref[...] = row * N + cols


def _sc_scatter_body(val_hbm, idx_hbm, out_hbm, zbuf, vbuf, ibuf, sem):
    cid = lax.axis_index("c")
    sid = lax.axis_index("s")
    wid = sid * 2 + cid
    base_n = wid * ROWS_PER_W
    slab_base = wid * SLAB

    zeros16 = jnp.zeros((16,), jnp.float32)

    def zero_vmem(i, carry):
        zbuf[pl.ds(i * 16, 16)] = zeros16
        return carry

    lax.fori_loop(0, ZW // 16, zero_vmem, 0)

    def zero_out(i, carry):
        pltpu.sync_copy(zbuf, out_hbm.at[pl.ds(slab_base + i * ZW, ZW)])
        return carry

    lax.fori_loop(0, SLAB // ZW, zero_out, 0)

    pltpu.sync_copy(val_hbm.at[:, pl.ds(base_n, ROWS_PER_W)], vbuf)
    pltpu.sync_copy(idx_hbm.at[:, pl.ds(base_n, ROWS_PER_W)], ibuf)

    def scatter(j, carry):
        pltpu.async_copy(vbuf.at[j], out_hbm.at[ibuf.at[j]], sem).wait()
        return carry

    lax.fori_loop(0, K, scatter, 0)


@jax.jit
def _run(coor, cols, w0t, b0r, w1t, b1r, gr, br, w2t, b2r, p):
    sums = pl.pallas_call(
        _stats_body,
        grid=(NEB,),
        in_specs=[
            pl.BlockSpec((BE, 2 * LOC), lambda i: (i, 0)),
            pl.BlockSpec((2 * LOC, 128), lambda i: (0, 0)),
            pl.BlockSpec((1, 128), lambda i: (0, 0)),
            pl.BlockSpec((128, 64), lambda i: (0, 0)),
            pl.BlockSpec((1, 64), lambda i: (0, 0)),
        ],
        out_specs=pl.BlockSpec((2, 64), lambda i: (0, 0)),
        out_shape=jax.ShapeDtypeStruct((2, 64), jnp.float32),
    )(coor, w0t, b0r, w1t, b1r)

    lck = pl.pallas_call(
        _lck_body,
        grid=(NEB,),
        in_specs=[
            pl.BlockSpec((BE, 2 * LOC), lambda i: (i, 0)),
            pl.BlockSpec((2 * LOC, 128), lambda i: (0, 0)),
            pl.BlockSpec((1, 128), lambda i: (0, 0)),
            pl.BlockSpec((128, 64), lambda i: (0, 0)),
            pl.BlockSpec((1, 64), lambda i: (0, 0)),
            pl.BlockSpec((2, 64), lambda i: (0, 0)),
            pl.BlockSpec((1, 64), lambda i: (0, 0)),
            pl.BlockSpec((1, 64), lambda i: (0, 0)),
            pl.BlockSpec((64, 1), lambda i: (0, 0)),
            pl.BlockSpec((1, 1), lambda i: (0, 0)),
        ],
        out_specs=pl.BlockSpec((BE, 1), lambda i: (i, 0)),
        out_shape=jax.ShapeDtypeStruct((E, 1), jnp.float32),
    )(coor, w0t, b0r, w1t, b1r, sums, gr, br, w2t, b2r)

    lck_t = lck.reshape(N, K).T
    c6t = coor.reshape(N, K, 2 * LOC).transpose(2, 1, 0)
    cols_t = cols.reshape(N, K).T

    val_t, idx_t = pl.pallas_call(
        _att_body,
        grid=(NRB,),
        in_specs=[
            pl.BlockSpec((2 * LOC, K, BN), lambda i: (0, 0, i)),
            pl.BlockSpec((K, BN), lambda i: (0, i)),
            pl.BlockSpec((K, BN), lambda i: (0, i)),
            pl.BlockSpec(memory_space=pltpu.SMEM),
        ],
        out_specs=[
            pl.BlockSpec((K, BN), lambda i: (0, i)),
            pl.BlockSpec((K, BN), lambda i: (0, i)),
        ],
        out_shape=[
            jax.ShapeDtypeStruct((K, N), jnp.float32),
            jax.ShapeDtypeStruct((K, N), jnp.int32),
        ],
    )(c6t, cols_t, lck_t, p)

    scatter = pl.kernel(
        _sc_scatter_body,
        out_type=jax.ShapeDtypeStruct((N * N,), jnp.float32),
        mesh=plsc.VectorSubcoreMesh(core_axis_name="c", subcore_axis_name="s"),
        scratch_types=[
            pltpu.VMEM((ZW,), jnp.float32),
            pltpu.VMEM((K, ROWS_PER_W), jnp.float32),
            pltpu.VMEM((K, ROWS_PER_W), jnp.int32),
            pltpu.SemaphoreType.DMA,
        ],
    )
    out = scatter(val_t, idx_t)
    return out.reshape(N, N)


def kernel(x, coor, sparse_idx, Wq, bq, Wk, bk, W0, b0, W1, b1, gamma, beta,
           W2, b2):
    cols = sparse_idx[1]
    amat = Wq.T @ Wk
    u = Wq.T @ bk
    v = Wk.T @ bq
    w = jnp.dot(bq, bk)
    p = jnp.concatenate([amat.reshape(-1), u, v, w[None]]).astype(jnp.float32)
    return _run(coor, cols, W0.T, b0.reshape(1, 128), W1.T, b1.reshape(1, 64),
                gamma.reshape(1, 64), beta.reshape(1, 64), W2.T,
                b2.reshape(1, 1), p)


# speedup vs baseline: 5.2576x; 2.7871x over previous
import jax
import jax.numpy as jnp
from jax import lax
from jax.experimental import pallas as pl
from jax.experimental.pallas import tpu as pltpu
from jax.experimental.pallas import tpu_sc as plsc

N = 4096
K = 32
E = N * K
LOC = 3
GEODESIC = 0.5

BE = 4096
NEB = E // BE
BN = 512
NRB = N // BN

ROWS_PER_W = 128
CH = 8
NCH = ROWS_PER_W // CH


def _mlp_t(ct, w0_ref, b0_ref, w1_ref, b1_ref):
    h = jnp.tanh(ct)
    h = jnp.dot(w0_ref[...], h, preferred_element_type=jnp.float32) + b0_ref[...]
    h = jnp.dot(w1_ref[...], h, preferred_element_type=jnp.float32) + b1_ref[...]
    return jnp.tanh(h)


def _stats_body(c_ref, w0_ref, b0_ref, w1_ref, b1_ref, o_ref):
    t = _mlp_t(c_ref[...], w0_ref, b0_ref, w1_ref, b1_ref)
    ps = jnp.sum(t, axis=1, keepdims=True)
    ps2 = jnp.sum(t * t, axis=1, keepdims=True)

    @pl.when(pl.program_id(0) == 0)
    def _():
        o_ref[...] = jnp.zeros_like(o_ref)

    o_ref[...] += jnp.concatenate([ps, ps2], axis=1)


def _lck_body(c_ref, w0_ref, b0_ref, w1_ref, b1_ref, s_ref, g_ref, be_ref,
              w2_ref, b2_ref, o_ref):
    t = _mlp_t(c_ref[...], w0_ref, b0_ref, w1_ref, b1_ref)
    inv_e = jnp.float32(1.0 / E)
    m = s_ref[:, 0:1] * inv_e
    var = s_ref[:, 1:2] * inv_e - m * m
    scale = g_ref[...] * lax.rsqrt(var + 1e-5)
    shift = be_ref[...] - m * scale
    g = jnp.tanh(t * scale + shift)
    lck = jnp.dot(w2_ref[...], g, preferred_element_type=jnp.float32) + b2_ref[...]
    o_ref[...] = jnp.maximum(lck, 0.0)[None]


def _att_body(c6_ref, cols_ref, lck_ref, p_ref, val_ref):
    c = c6_ref[...]
    s0 = c[0] + c[3]
    s1 = c[1] + c[4]
    s2 = c[2] + c[5]
    ctr0 = s0[0:1, :]
    ctr1 = s1[0:1, :]
    ctr2 = s2[0:1, :]
    ca0 = ctr0 * p_ref[0] + ctr1 * p_ref[3] + ctr2 * p_ref[6] + p_ref[12]
    ca1 = ctr0 * p_ref[1] + ctr1 * p_ref[4] + ctr2 * p_ref[7] + p_ref[13]
    ca2 = ctr0 * p_ref[2] + ctr1 * p_ref[5] + ctr2 * p_ref[8] + p_ref[14]
    cu = ctr0 * p_ref[9] + ctr1 * p_ref[10] + ctr2 * p_ref[11] + p_ref[15]
    att = ca0 * s0 + ca1 * s1 + ca2 * s2 + cu
    alpha = jnp.abs(att)

    cols = cols_ref[...]
    lck = lck_ref[...]
    lsum = jnp.zeros_like(lck)
    asum = jnp.zeros_like(lck)
    for k2 in range(K):
        m = cols == cols[k2:k2 + 1, :]
        lsum += jnp.where(m, lck[k2:k2 + 1, :], 0.0)
        asum += jnp.where(m, alpha[k2:k2 + 1, :], 0.0)

    val_ref[...] = lsum * jnp.exp(asum * jnp.float32(-GEODESIC))


def _sc_scatter_body(val_hbm, cols_hbm, out_hbm, vbuf, cbuf, dbuf0, dbuf1,
                     sem0, sem1):
    cid = lax.axis_index("c")
    sid = lax.axis_index("s")
    wid = sid * 2 + cid
    base = wid * ROWS_PER_W

    pltpu.sync_copy(val_hbm.at[:, pl.ds(base, ROWS_PER_W)], vbuf)
    pltpu.sync_copy(cols_hbm.at[:, pl.ds(base, ROWS_PER_W)], cbuf)

    z16 = jnp.zeros((16,), jnp.float32)
    for r in range(CH):
        def zrow(i, carry, _r=r):
            dbuf0[_r, pl.ds(i * 16, 16)] = z16
            dbuf1[_r, pl.ds(i * 16, 16)] = z16
            return carry
        lax.fori_loop(0, N // 16, zrow, 0)

    iota = lax.iota(jnp.int32, 16)
    half = [iota, iota + 16]

    def scatter_row(j, carry, db, ch):
        n_loc = ch * CH + j
        cful = jnp.zeros((16,), jnp.int32) + n_loc
        jful = jnp.zeros((16,), jnp.int32) + j
        for ri in half:
            c16 = plsc.load_gather(cbuf, [ri, cful])
            v16 = plsc.load_gather(vbuf, [ri, cful])
            plsc.store_scatter(db, [jful, c16], v16)
        return carry

    def rezero_row(j, carry, db, ch):
        n_loc = ch * CH + j
        cful = jnp.zeros((16,), jnp.int32) + n_loc
        jful = jnp.zeros((16,), jnp.int32) + j
        for ri in half:
            c16 = plsc.load_gather(cbuf, [ri, cful])
            plsc.store_scatter(db, [jful, c16], z16)
        return carry

    for ch in range(NCH):
        db = dbuf0 if ch % 2 == 0 else dbuf1
        sem = sem0 if ch % 2 == 0 else sem1
        if ch >= 2:
            pltpu.make_async_copy(
                db, out_hbm.at[pl.ds(base + (ch - 2) * CH, CH)], sem).wait()
            lax.fori_loop(0, CH,
                          lambda j, c, _db=db, _ch=ch - 2: rezero_row(j, c, _db, _ch),
                          0)
        lax.fori_loop(0, CH,
                      lambda j, c, _db=db, _ch=ch: scatter_row(j, c, _db, _ch),
                      0)
        pltpu.async_copy(db, out_hbm.at[pl.ds(base + ch * CH, CH)], sem)

    for ch in (NCH - 2, NCH - 1):
        db = dbuf0 if ch % 2 == 0 else dbuf1
        sem = sem0 if ch % 2 == 0 else sem1
        pltpu.make_async_copy(
            db, out_hbm.at[pl.ds(base + ch * CH, CH)], sem).wait()


@jax.jit
def _run(coor_t, cols, w0, b0c, w1, b1c, gc, bc, w2, b2c, p):
    sums = pl.pallas_call(
        _stats_body,
        grid=(NEB,),
        in_specs=[
            pl.BlockSpec((2 * LOC, BE), lambda i: (0, i)),
            pl.BlockSpec((128, 2 * LOC), lambda i: (0, 0)),
            pl.BlockSpec((128, 1), lambda i: (0, 0)),
            pl.BlockSpec((64, 128), lambda i: (0, 0)),
            pl.BlockSpec((64, 1), lambda i: (0, 0)),
        ],
        out_specs=pl.BlockSpec((64, 2), lambda i: (0, 0)),
        out_shape=jax.ShapeDtypeStruct((64, 2), jnp.float32),
    )(coor_t, w0, b0c, w1, b1c)

    lck = pl.pallas_call(
        _lck_body,
        grid=(NEB,),
        in_specs=[
            pl.BlockSpec((2 * LOC, BE), lambda i: (0, i)),
            pl.BlockSpec((128, 2 * LOC), lambda i: (0, 0)),
            pl.BlockSpec((128, 1), lambda i: (0, 0)),
            pl.BlockSpec((64, 128), lambda i: (0, 0)),
            pl.BlockSpec((64, 1), lambda i: (0, 0)),
            pl.BlockSpec((64, 2), lambda i: (0, 0)),
            pl.BlockSpec((64, 1), lambda i: (0, 0)),
            pl.BlockSpec((64, 1), lambda i: (0, 0)),
            pl.BlockSpec((1, 64), lambda i: (0, 0)),
            pl.BlockSpec((1, 1), lambda i: (0, 0)),
        ],
        out_specs=pl.BlockSpec((1, 1, BE), lambda i: (i, 0, 0)),
        out_shape=jax.ShapeDtypeStruct((NEB, 1, BE), jnp.float32),
    )(coor_t, w0, b0c, w1, b1c, sums, gc, bc, w2, b2c)

    c6t = coor_t.reshape(2 * LOC, N, K).transpose(0, 2, 1)
    cols_t = cols.reshape(N, K).T
    lck_t = lck.reshape(N, K).T

    val_t = pl.pallas_call(
        _att_body,
        grid=(NRB,),
        in_specs=[
            pl.BlockSpec((2 * LOC, K, BN), lambda i: (0, 0, i)),
            pl.BlockSpec((K, BN), lambda i: (0, i)),
            pl.BlockSpec((K, BN), lambda i: (0, i)),
            pl.BlockSpec(memory_space=pltpu.SMEM),
        ],
        out_specs=pl.BlockSpec((K, BN), lambda i: (0, i)),
        out_shape=jax.ShapeDtypeStruct((K, N), jnp.float32),
    )(c6t, cols_t, lck_t, p)

    scatter = pl.kernel(
        _sc_scatter_body,
        out_type=jax.ShapeDtypeStruct((N, N), jnp.float32),
        mesh=plsc.VectorSubcoreMesh(core_axis_name="c", subcore_axis_name="s"),
        compiler_params=pltpu.CompilerParams(needs_layout_passes=False),
        scratch_types=[
            pltpu.VMEM((K, ROWS_PER_W), jnp.float32),
            pltpu.VMEM((K, ROWS_PER_W), jnp.int32),
            pltpu.VMEM((CH, N), jnp.float32),
            pltpu.VMEM((CH, N), jnp.float32),
            pltpu.SemaphoreType.DMA,
            pltpu.SemaphoreType.DMA,
        ],
    )
    return scatter(val_t, cols_t)


def kernel(x, coor, sparse_idx, Wq, bq, Wk, bk, W0, b0, W1, b1, gamma, beta,
           W2, b2):
    cols = sparse_idx[1]
    amat = Wq.T @ Wk
    u = Wq.T @ bk
    v = Wk.T @ bq
    w = jnp.dot(bq, bk)
    p = jnp.concatenate([amat.reshape(-1), u, v, w[None]]).astype(jnp.float32)
    return _run(coor.T, cols, W0, b0.reshape(128, 1), W1, b1.reshape(64, 1),
                gamma.reshape(64, 1), beta.reshape(64, 1), W2,
                b2.reshape(1, 1), p)


# speedup vs baseline: 5.5499x; 1.0556x over previous
import jax
import jax.numpy as jnp
from jax import lax
from jax.experimental import pallas as pl
from jax.experimental.pallas import tpu as pltpu
from jax.experimental.pallas import tpu_sc as plsc

N = 4096
K = 32
E = N * K
LOC = 3
GEODESIC = 0.5

BE = 4096
NEB = E // BE
BN = 512
NRB = N // BN

ROWS_PER_W = 128
CH = 8
NCH = ROWS_PER_W // CH


def _mlp_body(c_ref, w0_ref, b0_ref, w1_ref, b1_ref, g_ref, be_ref,
              w2_ref, b2_ref, o_ref, t_ref, s_ref):
    ph = pl.program_id(0)
    i = pl.program_id(1)

    @pl.when(ph == 0)
    def _():
        h = jnp.tanh(c_ref[...])
        h = jnp.dot(w0_ref[...], h,
                    preferred_element_type=jnp.float32) + b0_ref[...]
        h = jnp.dot(w1_ref[...], h,
                    preferred_element_type=jnp.float32) + b1_ref[...]
        t = jnp.tanh(h)
        t_ref[:, pl.ds(i * BE, BE)] = t
        ps = jnp.sum(t, axis=1, keepdims=True)
        ps2 = jnp.sum(t * t, axis=1, keepdims=True)
        psum = jnp.concatenate([ps, ps2], axis=1)

        @pl.when(i == 0)
        def _():
            s_ref[...] = jnp.zeros_like(s_ref)

        s_ref[...] += psum
        o_ref[...] = jnp.zeros_like(o_ref)

    @pl.when(ph == 1)
    def _():
        t = t_ref[:, pl.ds(i * BE, BE)]
        inv_e = jnp.float32(1.0 / E)
        m = s_ref[:, 0:1] * inv_e
        var = s_ref[:, 1:2] * inv_e - m * m
        scale = g_ref[...] * lax.rsqrt(var + 1e-5)
        shift = be_ref[...] - m * scale
        g = jnp.tanh(t * scale + shift)
        lck = jnp.dot(w2_ref[...], g,
                      preferred_element_type=jnp.float32) + b2_ref[...]
        o_ref[...] = jnp.maximum(lck, 0.0)[None]


def _att_body(c6_ref, cols_ref, lck_ref, p_ref, val_ref):
    c = c6_ref[...]
    s0 = c[0] + c[3]
    s1 = c[1] + c[4]
    s2 = c[2] + c[5]
    ctr0 = s0[0:1, :]
    ctr1 = s1[0:1, :]
    ctr2 = s2[0:1, :]
    ca0 = ctr0 * p_ref[0] + ctr1 * p_ref[3] + ctr2 * p_ref[6] + p_ref[12]
    ca1 = ctr0 * p_ref[1] + ctr1 * p_ref[4] + ctr2 * p_ref[7] + p_ref[13]
    ca2 = ctr0 * p_ref[2] + ctr1 * p_ref[5] + ctr2 * p_ref[8] + p_ref[14]
    cu = ctr0 * p_ref[9] + ctr1 * p_ref[10] + ctr2 * p_ref[11] + p_ref[15]
    att = ca0 * s0 + ca1 * s1 + ca2 * s2 + cu
    alpha = jnp.abs(att)

    cols = cols_ref[...]
    lck = lck_ref[...]
    lsum = jnp.zeros_like(lck)
    asum = jnp.zeros_like(lck)
    for k2 in range(K):
        m = cols == cols[k2:k2 + 1, :]
        lsum += jnp.where(m, lck[k2:k2 + 1, :], 0.0)
        asum += jnp.where(m, alpha[k2:k2 + 1, :], 0.0)

    val_ref[...] = lsum * jnp.exp(asum * jnp.float32(-GEODESIC))


def _sc_scatter_body(val_hbm, cols_hbm, out_hbm, vbuf, cbuf, dbuf0, dbuf1,
                     sem0, sem1):
    cid = lax.axis_index("c")
    sid = lax.axis_index("s")
    wid = sid * 2 + cid
    base = wid * ROWS_PER_W

    pltpu.sync_copy(val_hbm.at[:, pl.ds(base, ROWS_PER_W)], vbuf)
    pltpu.sync_copy(cols_hbm.at[:, pl.ds(base, ROWS_PER_W)], cbuf)

    z16 = jnp.zeros((16,), jnp.float32)
    for r in range(CH):
        def zrow(i, carry, _r=r):
            dbuf0[_r, pl.ds(i * 16, 16)] = z16
            dbuf1[_r, pl.ds(i * 16, 16)] = z16
            return carry
        lax.fori_loop(0, N // 16, zrow, 0)

    iota = lax.iota(jnp.int32, 16)
    half = [iota, iota + 16]

    def scatter_row(j, carry, db, ch):
        n_loc = ch * CH + j
        cful = jnp.zeros((16,), jnp.int32) + n_loc
        jful = jnp.zeros((16,), jnp.int32) + j
        for ri in half:
            c16 = plsc.load_gather(cbuf, [ri, cful])
            v16 = plsc.load_gather(vbuf, [ri, cful])
            plsc.store_scatter(db, [jful, c16], v16)
        return carry

    def rezero_row(j, carry, db, ch):
        n_loc = ch * CH + j
        cful = jnp.zeros((16,), jnp.int32) + n_loc
        jful = jnp.zeros((16,), jnp.int32) + j
        for ri in half:
            c16 = plsc.load_gather(cbuf, [ri, cful])
            plsc.store_scatter(db, [jful, c16], z16)
        return carry

    for ch in range(NCH):
        db = dbuf0 if ch % 2 == 0 else dbuf1
        sem = sem0 if ch % 2 == 0 else sem1
        if ch >= 2:
            pltpu.make_async_copy(
                db, out_hbm.at[pl.ds(base + (ch - 2) * CH, CH)], sem).wait()
            lax.fori_loop(0, CH,
                          lambda j, c, _db=db, _ch=ch - 2: rezero_row(j, c, _db, _ch),
                          0)
        lax.fori_loop(0, CH,
                      lambda j, c, _db=db, _ch=ch: scatter_row(j, c, _db, _ch),
                      0)
        pltpu.async_copy(db, out_hbm.at[pl.ds(base + ch * CH, CH)], sem)

    for ch in (NCH - 2, NCH - 1):
        db = dbuf0 if ch % 2 == 0 else dbuf1
        sem = sem0 if ch % 2 == 0 else sem1
        pltpu.make_async_copy(
            db, out_hbm.at[pl.ds(base + ch * CH, CH)], sem).wait()


@jax.jit
def _run(coor, cols, w0, b0c, w1, b1c, gc, bc, w2, b2c, p):
    c6 = coor.reshape(N, K, 2 * LOC).transpose(2, 1, 0)
    c6e = c6.reshape(2 * LOC, E)
    cols_t = cols.reshape(N, K).T

    lck3 = pl.pallas_call(
        _mlp_body,
        grid=(2, NEB),
        in_specs=[
            pl.BlockSpec((2 * LOC, BE), lambda ph, i: (0, i)),
            pl.BlockSpec((128, 2 * LOC), lambda ph, i: (0, 0)),
            pl.BlockSpec((128, 1), lambda ph, i: (0, 0)),
            pl.BlockSpec((64, 128), lambda ph, i: (0, 0)),
            pl.BlockSpec((64, 1), lambda ph, i: (0, 0)),
            pl.BlockSpec((64, 1), lambda ph, i: (0, 0)),
            pl.BlockSpec((64, 1), lambda ph, i: (0, 0)),
            pl.BlockSpec((1, 64), lambda ph, i: (0, 0)),
            pl.BlockSpec((1, 1), lambda ph, i: (0, 0)),
        ],
        out_specs=pl.BlockSpec((1, 1, BE), lambda ph, i: (i, 0, 0)),
        out_shape=jax.ShapeDtypeStruct((K, 1, N), jnp.float32),
        scratch_shapes=[
            pltpu.VMEM((64, E), jnp.float32),
            pltpu.VMEM((64, 2), jnp.float32),
        ],
    )(c6e, w0, b0c, w1, b1c, gc, bc, w2, b2c)

    lck_t = lck3.reshape(K, N)

    val_t = pl.pallas_call(
        _att_body,
        grid=(NRB,),
        in_specs=[
            pl.BlockSpec((2 * LOC, K, BN), lambda i: (0, 0, i)),
            pl.BlockSpec((K, BN), lambda i: (0, i)),
            pl.BlockSpec((K, BN), lambda i: (0, i)),
            pl.BlockSpec(memory_space=pltpu.SMEM),
        ],
        out_specs=pl.BlockSpec((K, BN), lambda i: (0, i)),
        out_shape=jax.ShapeDtypeStruct((K, N), jnp.float32),
    )(c6, cols_t, lck_t, p)

    scatter = pl.kernel(
        _sc_scatter_body,
        out_type=jax.ShapeDtypeStruct((N, N), jnp.float32),
        mesh=plsc.VectorSubcoreMesh(core_axis_name="c", subcore_axis_name="s"),
        compiler_params=pltpu.CompilerParams(needs_layout_passes=False),
        scratch_types=[
            pltpu.VMEM((K, ROWS_PER_W), jnp.float32),
            pltpu.VMEM((K, ROWS_PER_W), jnp.int32),
            pltpu.VMEM((CH, N), jnp.float32),
            pltpu.VMEM((CH, N), jnp.float32),
            pltpu.SemaphoreType.DMA,
            pltpu.SemaphoreType.DMA,
        ],
    )
    return scatter(val_t, cols_t)


def kernel(x, coor, sparse_idx, Wq, bq, Wk, bk, W0, b0, W1, b1, gamma, beta,
           W2, b2):
    cols = sparse_idx[1]
    amat = Wq.T @ Wk
    u = Wq.T @ bk
    v = Wk.T @ bq
    w = jnp.dot(bq, bk)
    p = jnp.concatenate([amat.reshape(-1), u, v, w[None]]).astype(jnp.float32)
    return _run(coor, cols, W0, b0.reshape(128, 1), W1, b1.reshape(64, 1),
                gamma.reshape(64, 1), beta.reshape(64, 1), W2,
                b2.reshape(1, 1), p)


# speedup vs baseline: 5.6961x; 1.0263x over previous
import jax
import jax.numpy as jnp
from jax import lax
from jax.experimental import pallas as pl
from jax.experimental.pallas import tpu as pltpu
from jax.experimental.pallas import tpu_sc as plsc

N = 4096
K = 32
E = N * K
LOC = 3
GEODESIC = 0.5

BE = 4096
NEB = E // BE
BN = 512
NRB = N // BN

ROWS_PER_W = 128
CH = 8
NCH = ROWS_PER_W // CH


def _mlp_body(c_ref, w0_ref, w1_ref, w2_ref, wc_ref, o_ref, t_ref, s_ref):
    ph = pl.program_id(0)
    i = pl.program_id(1)

    @pl.when(ph == 0)
    def _():
        h = jnp.tanh(c_ref[...])
        h = jnp.dot(w0_ref[...], h,
                    preferred_element_type=jnp.float32) + wc_ref[0:128, :]
        h = jnp.dot(w1_ref[...], h,
                    preferred_element_type=jnp.float32) + wc_ref[128:192, :]
        t = jnp.tanh(h)
        t_ref[:, pl.ds(i * BE, BE)] = t
        ps = jnp.sum(t, axis=1, keepdims=True)
        ps2 = jnp.sum(t * t, axis=1, keepdims=True)
        psum = jnp.concatenate([ps, ps2], axis=1)

        @pl.when(i == 0)
        def _():
            s_ref[...] = jnp.zeros_like(s_ref)

        s_ref[...] += psum
        o_ref[...] = jnp.zeros_like(o_ref)

    @pl.when(ph == 1)
    def _():
        t = t_ref[:, pl.ds(i * BE, BE)]
        inv_e = jnp.float32(1.0 / E)
        m = s_ref[:, 0:1] * inv_e
        var = s_ref[:, 1:2] * inv_e - m * m
        scale = wc_ref[192:256, :] * lax.rsqrt(var + 1e-5)
        shift = wc_ref[256:320, :] - m * scale
        g = jnp.tanh(t * scale + shift)
        lck = jnp.dot(w2_ref[...], g,
                      preferred_element_type=jnp.float32) + wc_ref[320:321, :]
        o_ref[...] = jnp.maximum(lck, 0.0)[None]


def _att_body(c6_ref, cols_ref, lck_ref, p_ref, val_ref):
    c = c6_ref[...]
    s0 = c[0] + c[3]
    s1 = c[1] + c[4]
    s2 = c[2] + c[5]
    ctr0 = s0[0:1, :]
    ctr1 = s1[0:1, :]
    ctr2 = s2[0:1, :]
    ca0 = ctr0 * p_ref[0] + ctr1 * p_ref[3] + ctr2 * p_ref[6] + p_ref[12]
    ca1 = ctr0 * p_ref[1] + ctr1 * p_ref[4] + ctr2 * p_ref[7] + p_ref[13]
    ca2 = ctr0 * p_ref[2] + ctr1 * p_ref[5] + ctr2 * p_ref[8] + p_ref[14]
    cu = ctr0 * p_ref[9] + ctr1 * p_ref[10] + ctr2 * p_ref[11] + p_ref[15]
    att = ca0 * s0 + ca1 * s1 + ca2 * s2 + cu
    alpha = jnp.abs(att)

    cols = cols_ref[...]
    lck = lck_ref[...]
    lsum = jnp.zeros_like(lck)
    asum = jnp.zeros_like(lck)
    for k2 in range(K):
        m = cols == cols[k2:k2 + 1, :]
        lsum += jnp.where(m, lck[k2:k2 + 1, :], 0.0)
        asum += jnp.where(m, alpha[k2:k2 + 1, :], 0.0)

    val_ref[...] = lsum * jnp.exp(asum * jnp.float32(-GEODESIC))


def _sc_scatter_body(val_hbm, cols_hbm, out_hbm, vbuf, cbuf, dbuf0, dbuf1,
                     sem0, sem1):
    cid = lax.axis_index("c")
    sid = lax.axis_index("s")
    wid = sid * 2 + cid
    base = wid * ROWS_PER_W

    pltpu.sync_copy(val_hbm.at[:, pl.ds(base, ROWS_PER_W)], vbuf)
    pltpu.sync_copy(cols_hbm.at[:, pl.ds(base, ROWS_PER_W)], cbuf)

    z16 = jnp.zeros((16,), jnp.float32)
    for r in range(CH):
        def zrow(i, carry, _r=r):
            dbuf0[_r, pl.ds(i * 16, 16)] = z16
            dbuf1[_r, pl.ds(i * 16, 16)] = z16
            return carry
        lax.fori_loop(0, N // 16, zrow, 0)

    iota = lax.iota(jnp.int32, 16)
    half = [iota, iota + 16]

    def scatter_row(j, carry, db, ch):
        n_loc = ch * CH + j
        cful = jnp.zeros((16,), jnp.int32) + n_loc
        jful = jnp.zeros((16,), jnp.int32) + j
        for ri in half:
            c16 = plsc.load_gather(cbuf, [ri, cful])
            v16 = plsc.load_gather(vbuf, [ri, cful])
            plsc.store_scatter(db, [jful, c16], v16)
        return carry

    def rezero_row(j, carry, db, ch):
        n_loc = ch * CH + j
        cful = jnp.zeros((16,), jnp.int32) + n_loc
        jful = jnp.zeros((16,), jnp.int32) + j
        for ri in half:
            c16 = plsc.load_gather(cbuf, [ri, cful])
            plsc.store_scatter(db, [jful, c16], z16)
        return carry

    for ch in range(NCH):
        db = dbuf0 if ch % 2 == 0 else dbuf1
        sem = sem0 if ch % 2 == 0 else sem1
        if ch >= 2:
            pltpu.make_async_copy(
                db, out_hbm.at[pl.ds(base + (ch - 2) * CH, CH)], sem).wait()
            lax.fori_loop(0, CH,
                          lambda j, c, _db=db, _ch=ch - 2: rezero_row(j, c, _db, _ch),
                          0)
        lax.fori_loop(0, CH,
                      lambda j, c, _db=db, _ch=ch: scatter_row(j, c, _db, _ch),
                      0)
        pltpu.async_copy(db, out_hbm.at[pl.ds(base + ch * CH, CH)], sem)

    for ch in (NCH - 2, NCH - 1):
        db = dbuf0 if ch % 2 == 0 else dbuf1
        sem = sem0 if ch % 2 == 0 else sem1
        pltpu.make_async_copy(
            db, out_hbm.at[pl.ds(base + ch * CH, CH)], sem).wait()


@jax.jit
def _run(coor, cols, Wq, bq, Wk, bk, w0, b0, w1, b1, gamma, beta, w2, b2):
    amat = Wq.T @ Wk
    u = Wq.T @ bk
    v = Wk.T @ bq
    w = jnp.dot(bq, bk)
    p = jnp.concatenate([amat.reshape(-1), u, v, w[None]])
    wcol = jnp.concatenate([b0, b1, gamma, beta, b2])[:, None]

    c6 = coor.reshape(N, K, 2 * LOC).transpose(2, 1, 0)
    c6e = c6.reshape(2 * LOC, E)
    cols_t = cols.reshape(N, K).T

    lck3 = pl.pallas_call(
        _mlp_body,
        grid=(2, NEB),
        in_specs=[
            pl.BlockSpec((2 * LOC, BE), lambda ph, i: (0, i)),
            pl.BlockSpec((128, 2 * LOC), lambda ph, i: (0, 0)),
            pl.BlockSpec((64, 128), lambda ph, i: (0, 0)),
            pl.BlockSpec((1, 64), lambda ph, i: (0, 0)),
            pl.BlockSpec((321, 1), lambda ph, i: (0, 0)),
        ],
        out_specs=pl.BlockSpec((1, 1, BE), lambda ph, i: (i, 0, 0)),
        out_shape=jax.ShapeDtypeStruct((K, 1, N), jnp.float32),
        scratch_shapes=[
            pltpu.VMEM((64, E), jnp.float32),
            pltpu.VMEM((64, 2), jnp.float32),
        ],
    )(c6e, w0, w1, w2, wcol)

    lck_t = lck3.reshape(K, N)

    val_t = pl.pallas_call(
        _att_body,
        grid=(NRB,),
        in_specs=[
            pl.BlockSpec((2 * LOC, K, BN), lambda i: (0, 0, i)),
            pl.BlockSpec((K, BN), lambda i: (0, i)),
            pl.BlockSpec((K, BN), lambda i: (0, i)),
            pl.BlockSpec(memory_space=pltpu.SMEM),
        ],
        out_specs=pl.BlockSpec((K, BN), lambda i: (0, i)),
        out_shape=jax.ShapeDtypeStruct((K, N), jnp.float32),
    )(c6, cols_t, lck_t, p)

    scatter = pl.kernel(
        _sc_scatter_body,
        out_type=jax.ShapeDtypeStruct((N, N), jnp.float32),
        mesh=plsc.VectorSubcoreMesh(core_axis_name="c", subcore_axis_name="s"),
        compiler_params=pltpu.CompilerParams(needs_layout_passes=False),
        scratch_types=[
            pltpu.VMEM((K, ROWS_PER_W), jnp.float32),
            pltpu.VMEM((K, ROWS_PER_W), jnp.int32),
            pltpu.VMEM((CH, N), jnp.float32),
            pltpu.VMEM((CH, N), jnp.float32),
            pltpu.SemaphoreType.DMA,
            pltpu.SemaphoreType.DMA,
        ],
    )
    return scatter(val_t, cols_t)


def kernel(x, coor, sparse_idx, Wq, bq, Wk, bk, W0, b0, W1, b1, gamma, beta,
           W2, b2):
    return _run(coor, sparse_idx[1], Wq, bq, Wk, bk, W0, b0, W1, b1,
                gamma, beta, W2, b2)


# speedup vs baseline: 5.6983x; 1.0004x over previous
import jax
import jax.numpy as jnp
from jax import lax
from jax.experimental import pallas as pl
from jax.experimental.pallas import tpu as pltpu
from jax.experimental.pallas import tpu_sc as plsc

N = 4096
K = 32
E = N * K
LOC = 3
GEODESIC = 0.5

BE = 4096
NEB = E // BE
BN = 512
NRB = N // BN

ROWS_PER_W = 128
CH = 8
NCH = ROWS_PER_W // CH


def _mlp_body(c_ref, w0_ref, w1_ref, w2_ref, wc_ref, o_ref, t_ref, s_ref):
    ph = pl.program_id(0)
    i = pl.program_id(1)

    @pl.when(ph == 0)
    def _():
        h = jnp.tanh(c_ref[...])
        h = jnp.dot(w0_ref[...], h,
                    preferred_element_type=jnp.float32) + wc_ref[0:128, :]
        h = jnp.dot(w1_ref[...].astype(jnp.bfloat16), h.astype(jnp.bfloat16),
                    preferred_element_type=jnp.float32) + wc_ref[128:192, :]
        t = jnp.tanh(h)
        t_ref[:, pl.ds(i * BE, BE)] = t
        ps = jnp.sum(t, axis=1, keepdims=True)
        ps2 = jnp.sum(t * t, axis=1, keepdims=True)
        psum = jnp.concatenate([ps, ps2], axis=1)

        @pl.when(i == 0)
        def _():
            s_ref[...] = jnp.zeros_like(s_ref)

        s_ref[...] += psum
        o_ref[...] = jnp.zeros_like(o_ref)

    @pl.when(ph == 1)
    def _():
        t = t_ref[:, pl.ds(i * BE, BE)]
        inv_e = jnp.float32(1.0 / E)
        m = s_ref[:, 0:1] * inv_e
        var = s_ref[:, 1:2] * inv_e - m * m
        scale = wc_ref[192:256, :] * lax.rsqrt(var + 1e-5)
        shift = wc_ref[256:320, :] - m * scale
        g = jnp.tanh(t * scale + shift)
        lck = jnp.dot(w2_ref[...], g,
                      preferred_element_type=jnp.float32) + wc_ref[320:321, :]
        o_ref[...] = jnp.maximum(lck, 0.0)[None]


def _att_body(c6_ref, cols_ref, lck_ref, p_ref, val_ref):
    c = c6_ref[...]
    s0 = c[0] + c[3]
    s1 = c[1] + c[4]
    s2 = c[2] + c[5]
    ctr0 = s0[0:1, :]
    ctr1 = s1[0:1, :]
    ctr2 = s2[0:1, :]
    ca0 = ctr0 * p_ref[0] + ctr1 * p_ref[3] + ctr2 * p_ref[6] + p_ref[12]
    ca1 = ctr0 * p_ref[1] + ctr1 * p_ref[4] + ctr2 * p_ref[7] + p_ref[13]
    ca2 = ctr0 * p_ref[2] + ctr1 * p_ref[5] + ctr2 * p_ref[8] + p_ref[14]
    cu = ctr0 * p_ref[9] + ctr1 * p_ref[10] + ctr2 * p_ref[11] + p_ref[15]
    att = ca0 * s0 + ca1 * s1 + ca2 * s2 + cu
    alpha = jnp.abs(att)

    cols = cols_ref[...]
    lck = lck_ref[...]
    lsum = jnp.zeros_like(lck)
    asum = jnp.zeros_like(lck)
    for k2 in range(K):
        m = cols == cols[k2:k2 + 1, :]
        lsum += jnp.where(m, lck[k2:k2 + 1, :], 0.0)
        asum += jnp.where(m, alpha[k2:k2 + 1, :], 0.0)

    val_ref[...] = lsum * jnp.exp(asum * jnp.float32(-GEODESIC))


def _sc_scatter_body(val_hbm, cols_hbm, out_hbm, vbuf, cbuf, dbuf0, dbuf1,
                     sem0, sem1):
    cid = lax.axis_index("c")
    sid = lax.axis_index("s")
    wid = sid * 2 + cid
    base = wid * ROWS_PER_W

    pltpu.sync_copy(val_hbm.at[:, pl.ds(base, ROWS_PER_W)], vbuf)
    pltpu.sync_copy(cols_hbm.at[:, pl.ds(base, ROWS_PER_W)], cbuf)

    z16 = jnp.zeros((16,), jnp.float32)
    for r in range(CH):
        def zrow(i, carry, _r=r):
            dbuf0[_r, pl.ds(i * 16, 16)] = z16
            dbuf1[_r, pl.ds(i * 16, 16)] = z16
            return carry
        lax.fori_loop(0, N // 16, zrow, 0)

    iota = lax.iota(jnp.int32, 16)
    half = [iota, iota + 16]

    def scatter_row(j, carry, db, ch):
        n_loc = ch * CH + j
        cful = jnp.zeros((16,), jnp.int32) + n_loc
        jful = jnp.zeros((16,), jnp.int32) + j
        for ri in half:
            c16 = plsc.load_gather(cbuf, [ri, cful])
            v16 = plsc.load_gather(vbuf, [ri, cful])
            plsc.store_scatter(db, [jful, c16], v16)
        return carry

    def rezero_row(j, carry, db, ch):
        n_loc = ch * CH + j
        cful = jnp.zeros((16,), jnp.int32) + n_loc
        jful = jnp.zeros((16,), jnp.int32) + j
        for ri in half:
            c16 = plsc.load_gather(cbuf, [ri, cful])
            plsc.store_scatter(db, [jful, c16], z16)
        return carry

    for ch in range(NCH):
        db = dbuf0 if ch % 2 == 0 else dbuf1
        sem = sem0 if ch % 2 == 0 else sem1
        if ch >= 2:
            pltpu.make_async_copy(
                db, out_hbm.at[pl.ds(base + (ch - 2) * CH, CH)], sem).wait()
            lax.fori_loop(0, CH,
                          lambda j, c, _db=db, _ch=ch - 2: rezero_row(j, c, _db, _ch),
                          0)
        lax.fori_loop(0, CH,
                      lambda j, c, _db=db, _ch=ch: scatter_row(j, c, _db, _ch),
                      0)
        pltpu.async_copy(db, out_hbm.at[pl.ds(base + ch * CH, CH)], sem)

    for ch in (NCH - 2, NCH - 1):
        db = dbuf0 if ch % 2 == 0 else dbuf1
        sem = sem0 if ch % 2 == 0 else sem1
        pltpu.make_async_copy(
            db, out_hbm.at[pl.ds(base + ch * CH, CH)], sem).wait()


@jax.jit
def _run(coor, cols, Wq, bq, Wk, bk, w0, b0, w1, b1, gamma, beta, w2, b2):
    amat = Wq.T @ Wk
    u = Wq.T @ bk
    v = Wk.T @ bq
    w = jnp.dot(bq, bk)
    p = jnp.concatenate([amat.reshape(-1), u, v, w[None]])
    wcol = jnp.concatenate([b0, b1, gamma, beta, b2])[:, None]

    c6 = coor.reshape(N, K, 2 * LOC).transpose(2, 1, 0)
    c6e = c6.reshape(2 * LOC, E)
    cols_t = cols.reshape(N, K).T

    lck3 = pl.pallas_call(
        _mlp_body,
        grid=(2, NEB),
        in_specs=[
            pl.BlockSpec((2 * LOC, BE), lambda ph, i: (0, i)),
            pl.BlockSpec((128, 2 * LOC), lambda ph, i: (0, 0)),
            pl.BlockSpec((64, 128), lambda ph, i: (0, 0)),
            pl.BlockSpec((1, 64), lambda ph, i: (0, 0)),
            pl.BlockSpec((321, 1), lambda ph, i: (0, 0)),
        ],
        out_specs=pl.BlockSpec((1, 1, BE), lambda ph, i: (i, 0, 0)),
        out_shape=jax.ShapeDtypeStruct((K, 1, N), jnp.float32),
        scratch_shapes=[
            pltpu.VMEM((64, E), jnp.float32),
            pltpu.VMEM((64, 2), jnp.float32),
        ],
    )(c6e, w0, w1, w2, wcol)

    lck_t = lck3.reshape(K, N)

    val_t = pl.pallas_call(
        _att_body,
        grid=(NRB,),
        in_specs=[
            pl.BlockSpec((2 * LOC, K, BN), lambda i: (0, 0, i)),
            pl.BlockSpec((K, BN), lambda i: (0, i)),
            pl.BlockSpec((K, BN), lambda i: (0, i)),
            pl.BlockSpec(memory_space=pltpu.SMEM),
        ],
        out_specs=pl.BlockSpec((K, BN), lambda i: (0, i)),
        out_shape=jax.ShapeDtypeStruct((K, N), jnp.float32),
    )(c6, cols_t, lck_t, p)

    scatter = pl.kernel(
        _sc_scatter_body,
        out_type=jax.ShapeDtypeStruct((N, N), jnp.float32),
        mesh=plsc.VectorSubcoreMesh(core_axis_name="c", subcore_axis_name="s"),
        compiler_params=pltpu.CompilerParams(needs_layout_passes=False),
        scratch_types=[
            pltpu.VMEM((K, ROWS_PER_W), jnp.float32),
            pltpu.VMEM((K, ROWS_PER_W), jnp.int32),
            pltpu.VMEM((CH, N), jnp.float32),
            pltpu.VMEM((CH, N), jnp.float32),
            pltpu.SemaphoreType.DMA,
            pltpu.SemaphoreType.DMA,
        ],
    )
    return scatter(val_t, cols_t)


def kernel(x, coor, sparse_idx, Wq, bq, Wk, bk, W0, b0, W1, b1, gamma, beta,
           W2, b2):
    return _run(coor, sparse_idx[1], Wq, bq, Wk, bk, W0, b0, W1, b1,
                gamma, beta, W2, b2)


# speedup vs baseline: 6.3877x; 1.1210x over previous
import jax
import jax.numpy as jnp
from jax import lax
from jax.experimental import pallas as pl
from jax.experimental.pallas import tpu as pltpu
from jax.experimental.pallas import tpu_sc as plsc

N = 4096
K = 32
E = N * K
LOC = 3
GEODESIC = 0.5

BE = 8192
KPB = BE // N
NEB = E // BE
BN = 512
NRB = N // BN

ROWS_PER_W = 128
CH = 8
NCH = ROWS_PER_W // CH


def _mlp_body(c_ref, w0_ref, w1_ref, w2_ref, wc_ref, o_ref, t_ref, s_ref):
    ph = pl.program_id(0)
    i = pl.program_id(1)

    @pl.when(ph == 0)
    def _():
        h = jnp.tanh(c_ref[...])
        h = jnp.dot(w0_ref[...], h,
                    preferred_element_type=jnp.float32) + wc_ref[0:128, :]
        h = jnp.dot(w1_ref[...], h,
                    preferred_element_type=jnp.float32) + wc_ref[128:192, :]
        t = jnp.tanh(h)
        t_ref[:, pl.ds(i * BE, BE)] = t
        ps = jnp.sum(t, axis=1, keepdims=True)
        ps2 = jnp.sum(t * t, axis=1, keepdims=True)
        psum = jnp.concatenate([ps, ps2], axis=1)

        @pl.when(i == 0)
        def _():
            s_ref[...] = jnp.zeros_like(s_ref)

        s_ref[...] += psum
        o_ref[...] = jnp.zeros_like(o_ref)

    @pl.when(ph == 1)
    def _():
        t = t_ref[:, pl.ds(i * BE, BE)]
        inv_e = jnp.float32(1.0 / E)
        m = s_ref[:, 0:1] * inv_e
        var = s_ref[:, 1:2] * inv_e - m * m
        scale = wc_ref[192:256, :] * lax.rsqrt(var + 1e-5)
        shift = wc_ref[256:320, :] - m * scale
        g = jnp.tanh(t * scale + shift)
        lck = jnp.dot(w2_ref[...], g,
                      preferred_element_type=jnp.float32) + wc_ref[320:321, :]
        lck = jnp.maximum(lck, 0.0)
        for q in range(KPB):
            o_ref[q:q + 1] = lck[:, q * N:(q + 1) * N][None]


def _att_body(c6_ref, cols_ref, lck_ref, p_ref, val_ref):
    c = c6_ref[...]
    s0 = c[0] + c[3]
    s1 = c[1] + c[4]
    s2 = c[2] + c[5]
    ctr0 = s0[0:1, :]
    ctr1 = s1[0:1, :]
    ctr2 = s2[0:1, :]
    ca0 = ctr0 * p_ref[0] + ctr1 * p_ref[3] + ctr2 * p_ref[6] + p_ref[12]
    ca1 = ctr0 * p_ref[1] + ctr1 * p_ref[4] + ctr2 * p_ref[7] + p_ref[13]
    ca2 = ctr0 * p_ref[2] + ctr1 * p_ref[5] + ctr2 * p_ref[8] + p_ref[14]
    cu = ctr0 * p_ref[9] + ctr1 * p_ref[10] + ctr2 * p_ref[11] + p_ref[15]
    att = ca0 * s0 + ca1 * s1 + ca2 * s2 + cu
    alpha = jnp.abs(att)

    cols = cols_ref[...]
    lck = lck_ref[...]
    lsum = jnp.zeros_like(lck)
    asum = jnp.zeros_like(lck)
    for k2 in range(K):
        m = cols == cols[k2:k2 + 1, :]
        lsum += jnp.where(m, lck[k2:k2 + 1, :], 0.0)
        asum += jnp.where(m, alpha[k2:k2 + 1, :], 0.0)

    val_ref[...] = lsum * jnp.exp(asum * jnp.float32(-GEODESIC))


def _sc_scatter_body(val_hbm, cols_hbm, out_hbm, vbuf, cbuf, dbuf0, dbuf1,
                     sem0, sem1):
    cid = lax.axis_index("c")
    sid = lax.axis_index("s")
    wid = sid * 2 + cid
    base = wid * ROWS_PER_W

    pltpu.sync_copy(val_hbm.at[:, pl.ds(base, ROWS_PER_W)], vbuf)
    pltpu.sync_copy(cols_hbm.at[:, pl.ds(base, ROWS_PER_W)], cbuf)

    z16 = jnp.zeros((16,), jnp.float32)
    for r in range(CH):
        def zrow(i, carry, _r=r):
            dbuf0[_r, pl.ds(i * 16, 16)] = z16
            dbuf1[_r, pl.ds(i * 16, 16)] = z16
            return carry
        lax.fori_loop(0, N // 16, zrow, 0)

    iota = lax.iota(jnp.int32, 16)
    half = [iota, iota + 16]

    def scatter_row(j, carry, db, ch):
        n_loc = ch * CH + j
        cful = jnp.zeros((16,), jnp.int32) + n_loc
        jful = jnp.zeros((16,), jnp.int32) + j
        for ri in half:
            c16 = plsc.load_gather(cbuf, [ri, cful])
            v16 = plsc.load_gather(vbuf, [ri, cful])
            plsc.store_scatter(db, [jful, c16], v16)
        return carry

    def rezero_row(j, carry, db, ch):
        n_loc = ch * CH + j
        cful = jnp.zeros((16,), jnp.int32) + n_loc
        jful = jnp.zeros((16,), jnp.int32) + j
        for ri in half:
            c16 = plsc.load_gather(cbuf, [ri, cful])
            plsc.store_scatter(db, [jful, c16], z16)
        return carry

    for ch in range(NCH):
        db = dbuf0 if ch % 2 == 0 else dbuf1
        sem = sem0 if ch % 2 == 0 else sem1
        if ch >= 2:
            pltpu.make_async_copy(
                db, out_hbm.at[pl.ds(base + (ch - 2) * CH, CH)], sem).wait()
            lax.fori_loop(0, CH,
                          lambda j, c, _db=db, _ch=ch - 2: rezero_row(j, c, _db, _ch),
                          0)
        lax.fori_loop(0, CH,
                      lambda j, c, _db=db, _ch=ch: scatter_row(j, c, _db, _ch),
                      0)
        pltpu.async_copy(db, out_hbm.at[pl.ds(base + ch * CH, CH)], sem)

    for ch in (NCH - 2, NCH - 1):
        db = dbuf0 if ch % 2 == 0 else dbuf1
        sem = sem0 if ch % 2 == 0 else sem1
        pltpu.make_async_copy(
            db, out_hbm.at[pl.ds(base + ch * CH, CH)], sem).wait()


@jax.jit
def _run(coor, cols, Wq, bq, Wk, bk, w0, b0, w1, b1, gamma, beta, w2, b2):
    amat = Wq.T @ Wk
    u = Wq.T @ bk
    v = Wk.T @ bq
    w = jnp.dot(bq, bk)
    p = jnp.concatenate([amat.reshape(-1), u, v, w[None]])
    wcol = jnp.concatenate([b0, b1, gamma, beta, b2])[:, None]

    c6 = coor.reshape(N, K, 2 * LOC).transpose(2, 1, 0)
    c6e = c6.reshape(2 * LOC, E)
    cols_t = cols.reshape(N, K).T

    lck3 = pl.pallas_call(
        _mlp_body,
        grid=(2, NEB),
        in_specs=[
            pl.BlockSpec((2 * LOC, BE), lambda ph, i: (0, i)),
            pl.BlockSpec((128, 2 * LOC), lambda ph, i: (0, 0)),
            pl.BlockSpec((64, 128), lambda ph, i: (0, 0)),
            pl.BlockSpec((1, 64), lambda ph, i: (0, 0)),
            pl.BlockSpec((321, 1), lambda ph, i: (0, 0)),
        ],
        out_specs=pl.BlockSpec((KPB, 1, N), lambda ph, i: (i, 0, 0)),
        out_shape=jax.ShapeDtypeStruct((K, 1, N), jnp.float32),
        scratch_shapes=[
            pltpu.VMEM((64, E), jnp.float32),
            pltpu.VMEM((64, 2), jnp.float32),
        ],
    )(c6e, w0, w1, w2, wcol)

    lck_t = lck3.reshape(K, N)

    val_t = pl.pallas_call(
        _att_body,
        grid=(NRB,),
        in_specs=[
            pl.BlockSpec((2 * LOC, K, BN), lambda i: (0, 0, i)),
            pl.BlockSpec((K, BN), lambda i: (0, i)),
            pl.BlockSpec((K, BN), lambda i: (0, i)),
            pl.BlockSpec(memory_space=pltpu.SMEM),
        ],
        out_specs=pl.BlockSpec((K, BN), lambda i: (0, i)),
        out_shape=jax.ShapeDtypeStruct((K, N), jnp.float32),
    )(c6, cols_t, lck_t, p)

    scatter = pl.kernel(
        _sc_scatter_body,
        out_type=jax.ShapeDtypeStruct((N, N), jnp.float32),
        mesh=plsc.VectorSubcoreMesh(core_axis_name="c", subcore_axis_name="s"),
        compiler_params=pltpu.CompilerParams(needs_layout_passes=False),
        scratch_types=[
            pltpu.VMEM((K, ROWS_PER_W), jnp.float32),
            pltpu.VMEM((K, ROWS_PER_W), jnp.int32),
            pltpu.VMEM((CH, N), jnp.float32),
            pltpu.VMEM((CH, N), jnp.float32),
            pltpu.SemaphoreType.DMA,
            pltpu.SemaphoreType.DMA,
        ],
    )
    return scatter(val_t, cols_t)


def kernel(x, coor, sparse_idx, Wq, bq, Wk, bk, W0, b0, W1, b1, gamma, beta,
           W2, b2):
    return _run(coor, sparse_idx[1], Wq, bq, Wk, bk, W0, b0, W1, b1,
                gamma, beta, W2, b2)


# speedup vs baseline: 6.7387x; 1.0549x over previous
import jax
import jax.numpy as jnp
from jax import lax
from jax.experimental import pallas as pl
from jax.experimental.pallas import tpu as pltpu
from jax.experimental.pallas import tpu_sc as plsc

N = 4096
K = 32
E = N * K
LOC = 3
GEODESIC = 0.5

BE = 16384
KPB = BE // N
NEB = E // BE
BN = 512
NRB = N // BN

ROWS_PER_W = 128
CH = 8
NCH = ROWS_PER_W // CH


def _mlp_body(c_ref, w0_ref, w1_ref, w2_ref, wc_ref, o_ref, t_ref, s_ref):
    ph = pl.program_id(0)
    i = pl.program_id(1)

    @pl.when(ph == 0)
    def _():
        h = jnp.tanh(c_ref[...])
        h = jnp.dot(w0_ref[...], h,
                    preferred_element_type=jnp.float32) + wc_ref[0:128, :]
        h = jnp.dot(w1_ref[...], h,
                    preferred_element_type=jnp.float32) + wc_ref[128:192, :]
        t = jnp.tanh(h)
        t_ref[:, pl.ds(i * BE, BE)] = t
        ps = jnp.sum(t, axis=1, keepdims=True)
        ps2 = jnp.sum(t * t, axis=1, keepdims=True)
        psum = jnp.concatenate([ps, ps2], axis=1)

        @pl.when(i == 0)
        def _():
            s_ref[...] = jnp.zeros_like(s_ref)

        s_ref[...] += psum
        o_ref[...] = jnp.zeros_like(o_ref)

    @pl.when(ph == 1)
    def _():
        t = t_ref[:, pl.ds(i * BE, BE)]
        inv_e = jnp.float32(1.0 / E)
        m = s_ref[:, 0:1] * inv_e
        var = s_ref[:, 1:2] * inv_e - m * m
        scale = wc_ref[192:256, :] * lax.rsqrt(var + 1e-5)
        shift = wc_ref[256:320, :] - m * scale
        g = jnp.tanh(t * scale + shift)
        lck = jnp.dot(w2_ref[...], g,
                      preferred_element_type=jnp.float32) + wc_ref[320:321, :]
        lck = jnp.maximum(lck, 0.0)
        for q in range(KPB):
            o_ref[q:q + 1] = lck[:, q * N:(q + 1) * N][None]


def _att_body(c6_ref, cols_ref, lck_ref, p_ref, val_ref):
    c = c6_ref[...]
    s0 = c[0] + c[3]
    s1 = c[1] + c[4]
    s2 = c[2] + c[5]
    ctr0 = s0[0:1, :]
    ctr1 = s1[0:1, :]
    ctr2 = s2[0:1, :]
    ca0 = ctr0 * p_ref[0] + ctr1 * p_ref[3] + ctr2 * p_ref[6] + p_ref[12]
    ca1 = ctr0 * p_ref[1] + ctr1 * p_ref[4] + ctr2 * p_ref[7] + p_ref[13]
    ca2 = ctr0 * p_ref[2] + ctr1 * p_ref[5] + ctr2 * p_ref[8] + p_ref[14]
    cu = ctr0 * p_ref[9] + ctr1 * p_ref[10] + ctr2 * p_ref[11] + p_ref[15]
    att = ca0 * s0 + ca1 * s1 + ca2 * s2 + cu
    alpha = jnp.abs(att)

    cols = cols_ref[...]
    lck = lck_ref[...]
    lsum = jnp.zeros_like(lck)
    asum = jnp.zeros_like(lck)
    for k2 in range(K):
        m = cols == cols[k2:k2 + 1, :]
        lsum += jnp.where(m, lck[k2:k2 + 1, :], 0.0)
        asum += jnp.where(m, alpha[k2:k2 + 1, :], 0.0)

    val_ref[...] = lsum * jnp.exp(asum * jnp.float32(-GEODESIC))


def _sc_scatter_body(val_hbm, cols_hbm, out_hbm, vbuf, cbuf, dbuf0, dbuf1,
                     sem0, sem1):
    cid = lax.axis_index("c")
    sid = lax.axis_index("s")
    wid = sid * 2 + cid
    base = wid * ROWS_PER_W

    pltpu.sync_copy(val_hbm.at[:, pl.ds(base, ROWS_PER_W)], vbuf)
    pltpu.sync_copy(cols_hbm.at[:, pl.ds(base, ROWS_PER_W)], cbuf)

    z16 = jnp.zeros((16,), jnp.float32)
    for r in range(CH):
        def zrow(i, carry, _r=r):
            dbuf0[_r, pl.ds(i * 16, 16)] = z16
            dbuf1[_r, pl.ds(i * 16, 16)] = z16
            return carry
        lax.fori_loop(0, N // 16, zrow, 0)

    iota = lax.iota(jnp.int32, 16)
    half = [iota, iota + 16]

    def scatter_row(j, carry, db, ch):
        n_loc = ch * CH + j
        cful = jnp.zeros((16,), jnp.int32) + n_loc
        jful = jnp.zeros((16,), jnp.int32) + j
        for ri in half:
            c16 = plsc.load_gather(cbuf, [ri, cful])
            v16 = plsc.load_gather(vbuf, [ri, cful])
            plsc.store_scatter(db, [jful, c16], v16)
        return carry

    def rezero_row(j, carry, db, ch):
        n_loc = ch * CH + j
        cful = jnp.zeros((16,), jnp.int32) + n_loc
        jful = jnp.zeros((16,), jnp.int32) + j
        for ri in half:
            c16 = plsc.load_gather(cbuf, [ri, cful])
            plsc.store_scatter(db, [jful, c16], z16)
        return carry

    for ch in range(NCH):
        db = dbuf0 if ch % 2 == 0 else dbuf1
        sem = sem0 if ch % 2 == 0 else sem1
        if ch >= 2:
            pltpu.make_async_copy(
                db, out_hbm.at[pl.ds(base + (ch - 2) * CH, CH)], sem).wait()
            lax.fori_loop(0, CH,
                          lambda j, c, _db=db, _ch=ch - 2: rezero_row(j, c, _db, _ch),
                          0)
        lax.fori_loop(0, CH,
                      lambda j, c, _db=db, _ch=ch: scatter_row(j, c, _db, _ch),
                      0)
        pltpu.async_copy(db, out_hbm.at[pl.ds(base + ch * CH, CH)], sem)

    for ch in (NCH - 2, NCH - 1):
        db = dbuf0 if ch % 2 == 0 else dbuf1
        sem = sem0 if ch % 2 == 0 else sem1
        pltpu.make_async_copy(
            db, out_hbm.at[pl.ds(base + ch * CH, CH)], sem).wait()


@jax.jit
def _run(coor, cols, Wq, bq, Wk, bk, w0, b0, w1, b1, gamma, beta, w2, b2):
    amat = Wq.T @ Wk
    u = Wq.T @ bk
    v = Wk.T @ bq
    w = jnp.dot(bq, bk)
    p = jnp.concatenate([amat.reshape(-1), u, v, w[None]])
    wcol = jnp.concatenate([b0, b1, gamma, beta, b2])[:, None]

    c6 = coor.reshape(N, K, 2 * LOC).transpose(2, 1, 0)
    c6e = c6.reshape(2 * LOC, E)
    cols_t = cols.reshape(N, K).T

    lck3 = pl.pallas_call(
        _mlp_body,
        grid=(2, NEB),
        in_specs=[
            pl.BlockSpec((2 * LOC, BE), lambda ph, i: (0, i)),
            pl.BlockSpec((128, 2 * LOC), lambda ph, i: (0, 0)),
            pl.BlockSpec((64, 128), lambda ph, i: (0, 0)),
            pl.BlockSpec((1, 64), lambda ph, i: (0, 0)),
            pl.BlockSpec((321, 1), lambda ph, i: (0, 0)),
        ],
        out_specs=pl.BlockSpec((KPB, 1, N), lambda ph, i: (i, 0, 0)),
        out_shape=jax.ShapeDtypeStruct((K, 1, N), jnp.float32),
        scratch_shapes=[
            pltpu.VMEM((64, E), jnp.float32),
            pltpu.VMEM((64, 2), jnp.float32),
        ],
    )(c6e, w0, w1, w2, wcol)

    lck_t = lck3.reshape(K, N)

    val_t = pl.pallas_call(
        _att_body,
        grid=(NRB,),
        in_specs=[
            pl.BlockSpec((2 * LOC, K, BN), lambda i: (0, 0, i)),
            pl.BlockSpec((K, BN), lambda i: (0, i)),
            pl.BlockSpec((K, BN), lambda i: (0, i)),
            pl.BlockSpec(memory_space=pltpu.SMEM),
        ],
        out_specs=pl.BlockSpec((K, BN), lambda i: (0, i)),
        out_shape=jax.ShapeDtypeStruct((K, N), jnp.float32),
    )(c6, cols_t, lck_t, p)

    scatter = pl.kernel(
        _sc_scatter_body,
        out_type=jax.ShapeDtypeStruct((N, N), jnp.float32),
        mesh=plsc.VectorSubcoreMesh(core_axis_name="c", subcore_axis_name="s"),
        compiler_params=pltpu.CompilerParams(needs_layout_passes=False),
        scratch_types=[
            pltpu.VMEM((K, ROWS_PER_W), jnp.float32),
            pltpu.VMEM((K, ROWS_PER_W), jnp.int32),
            pltpu.VMEM((CH, N), jnp.float32),
            pltpu.VMEM((CH, N), jnp.float32),
            pltpu.SemaphoreType.DMA,
            pltpu.SemaphoreType.DMA,
        ],
    )
    return scatter(val_t, cols_t)


def kernel(x, coor, sparse_idx, Wq, bq, Wk, bk, W0, b0, W1, b1, gamma, beta,
           W2, b2):
    return _run(coor, sparse_idx[1], Wq, bq, Wk, bk, W0, b0, W1, b1,
                gamma, beta, W2, b2)
